# baseline (speedup 1.0000x reference)
import functools

import jax
import jax.numpy as jnp
from jax import lax
from jax.experimental import pallas as pl
from jax.experimental.pallas import tpu as pltpu
from jax.experimental.pallas import tpu_sc as plsc

_LAMBDA = 0.1
_STEP = 2.0 / 15.0


def _quantize(v):
    c = jnp.clip(v, -_LAMBDA, _LAMBDA)
    s = v - c
    t = jnp.clip(s * 7.5 + 8.0, 0.5, 15.5)
    f = t.astype(jnp.int32).astype(jnp.float32)
    q = f * _STEP - 1.0
    q = jnp.where(jnp.abs(s) < 1e-6, 0.0, q)
    return (q - v) + v


def _tc_body(x_ref, o_ref):
    o_ref[...] = _quantize(x_ref[...])


def _tc_call(x3, grid=8):
    b, r, c = x3.shape
    block = b // grid
    return pl.pallas_call(
        _tc_body,
        grid=(grid,),
        in_specs=[pl.BlockSpec((block, r, c), lambda i: (i, 0, 0))],
        out_specs=pl.BlockSpec((block, r, c), lambda i: (i, 0, 0)),
        out_shape=jax.ShapeDtypeStruct((b, r, c), x3.dtype),
    )(x3)


_NC, _NS, _L = 2, 16, 16
_NW = _NC * _NS


def _make_sc_call(n):
    per_w = n // _NW
    ch = min(per_w, 16384)
    nch = per_w // ch

    @functools.partial(
        pl.kernel,
        mesh=plsc.VectorSubcoreMesh(core_axis_name="c", subcore_axis_name="s"),
        out_type=jax.ShapeDtypeStruct((n,), jnp.float32),
        scratch_types=[
            pltpu.VMEM((ch,), jnp.float32),
            pltpu.VMEM((ch,), jnp.float32),
        ],
    )
    def sc_quantize(x_hbm, o_hbm, in_v, out_v):
        wid = lax.axis_index("s") * _NC + lax.axis_index("c")
        base = wid * per_w

        def do_chunk(ci, carry):
            off = base + ci * ch
            pltpu.sync_copy(x_hbm.at[pl.ds(off, ch)], in_v)

            def body(i, carry2):
                v = in_v[pl.ds(i * _L, _L)]
                out_v[pl.ds(i * _L, _L)] = _quantize(v)
                return carry2

            lax.fori_loop(0, ch // _L, body, 0)
            pltpu.sync_copy(out_v, o_hbm.at[pl.ds(off, ch)])
            return carry

        lax.fori_loop(0, nch, do_chunk, 0)

    return sc_quantize


_SC_CALL_CACHE = {}


def _sc_call(xf):
    n = xf.shape[0]
    if n not in _SC_CALL_CACHE:
        _SC_CALL_CACHE[n] = _make_sc_call(n)
    return _SC_CALL_CACHE[n](xf)


def kernel(x):
    return _tc_call(x, grid=16)


# speedup vs baseline: 1.0560x; 1.0560x over previous
import functools

import jax
import jax.numpy as jnp
from jax import lax
from jax.experimental import pallas as pl
from jax.experimental.pallas import tpu as pltpu
from jax.experimental.pallas import tpu_sc as plsc

_LAMBDA = 0.1
_STEP = 2.0 / 15.0


def _quantize(v):
    c = jnp.clip(v, -_LAMBDA, _LAMBDA)
    s = v - c
    t = jnp.clip(s * 7.5 + 8.0, 0.5, 15.5)
    f = t.astype(jnp.int32).astype(jnp.float32)
    q = f * _STEP - 1.0
    q = jnp.where(jnp.abs(s) < 1e-6, 0.0, q)
    return (q - v) + v


def _tc_body(x_ref, o_ref):
    o_ref[...] = _quantize(x_ref[...])


def _tc_call(x3, grid=8):
    b, r, c = x3.shape
    block = b // grid
    return pl.pallas_call(
        _tc_body,
        grid=(grid,),
        in_specs=[pl.BlockSpec((block, r, c), lambda i: (i, 0, 0))],
        out_specs=pl.BlockSpec((block, r, c), lambda i: (i, 0, 0)),
        out_shape=jax.ShapeDtypeStruct((b, r, c), x3.dtype),
    )(x3)


_NC, _NS, _L = 2, 16, 16
_NW = _NC * _NS


def _make_sc_call(n):
    per_w = n // _NW
    ch = min(per_w, 16384)
    nch = per_w // ch

    @functools.partial(
        pl.kernel,
        mesh=plsc.VectorSubcoreMesh(core_axis_name="c", subcore_axis_name="s"),
        out_type=jax.ShapeDtypeStruct((n,), jnp.float32),
        scratch_types=[
            pltpu.VMEM((ch,), jnp.float32),
            pltpu.VMEM((ch,), jnp.float32),
        ],
    )
    def sc_quantize(x_hbm, o_hbm, in_v, out_v):
        wid = lax.axis_index("s") * _NC + lax.axis_index("c")
        base = wid * per_w

        def do_chunk(ci, carry):
            off = base + ci * ch
            pltpu.sync_copy(x_hbm.at[pl.ds(off, ch)], in_v)

            def body(i, carry2):
                v = in_v[pl.ds(i * _L, _L)]
                out_v[pl.ds(i * _L, _L)] = _quantize(v)
                return carry2

            lax.fori_loop(0, ch // _L, body, 0)
            pltpu.sync_copy(out_v, o_hbm.at[pl.ds(off, ch)])
            return carry

        lax.fori_loop(0, nch, do_chunk, 0)

    return sc_quantize


_SC_CALL_CACHE = {}


def _sc_call(xf):
    n = xf.shape[0]
    if n not in _SC_CALL_CACHE:
        _SC_CALL_CACHE[n] = _make_sc_call(n)
    return _SC_CALL_CACHE[n](xf)


def kernel(x):
    return _tc_call(x, grid=4)
